# Initial kernel scaffold; baseline (speedup 1.0000x reference)
import jax
import jax.numpy as jnp
from jax.experimental import pallas as pl


def kernel(x, edge_types, mul_weight, bias_weight, bessel_weights, prefactor):
    raise NotImplementedError("write your pallas kernel here")


# speedup vs baseline: 22.1964x; 22.1964x over previous
import functools

import jax
import jax.numpy as jnp
from jax import lax
from jax.experimental import pallas as pl
from jax.experimental.pallas import tpu as pltpu
from jax.experimental.pallas import tpu_sc as plsc

NUM_BASIS = 16
LANES_PER_ROW = 128
EXP_COLS = LANES_PER_ROW * NUM_BASIS


def _sc_gather_call(et_flat, mul_tbl, bias_tbl, chunk, n_chunks, e_per_worker):
    E = et_flat.shape[0] // 2
    T = mul_tbl.shape[0]
    mesh = plsc.VectorSubcoreMesh(core_axis_name="c", subcore_axis_name="s")

    @functools.partial(
        pl.kernel,
        mesh=mesh,
        compiler_params=pltpu.CompilerParams(needs_layout_passes=False),
        out_type=[
            jax.ShapeDtypeStruct((E,), jnp.float32),
            jax.ShapeDtypeStruct((E,), jnp.float32),
        ],
        scratch_types=[
            pltpu.VMEM((2 * chunk,), jnp.int32),
            pltpu.VMEM((chunk,), jnp.float32),
            pltpu.VMEM((chunk,), jnp.float32),
            pltpu.VMEM((T,), jnp.float32),
            pltpu.VMEM((T,), jnp.float32),
        ],
    )
    def sc_kernel(et_hbm, mt_hbm, bt_hbm, mul_out, bias_out,
                  et_v, mul_v, bias_v, mt_v, bt_v):
        nc = 2
        wid = lax.axis_index("s") * nc + lax.axis_index("c")
        pltpu.sync_copy(mt_hbm, mt_v)
        pltpu.sync_copy(bt_hbm, bt_v)
        base_e = wid * e_per_worker
        even = lax.iota(jnp.int32, 16) * 2

        def chunk_body(ci, carry):
            e0 = base_e + ci * chunk
            pltpu.sync_copy(et_hbm.at[pl.ds(e0 * 2, 2 * chunk)], et_v)

            def grp(j, c2):
                off = j * 32
                i0 = plsc.load_gather(et_v, [even + off])
                i1 = plsc.load_gather(et_v, [even + (off + 1)])
                m = plsc.load_gather(mt_v, [i0]) + plsc.load_gather(mt_v, [i1])
                b = plsc.load_gather(bt_v, [i0]) + plsc.load_gather(bt_v, [i1])
                mul_v[pl.ds(j * 16, 16)] = m
                bias_v[pl.ds(j * 16, 16)] = b
                return c2

            lax.fori_loop(0, chunk // 16, grp, 0, unroll=4)
            pltpu.sync_copy(mul_v, mul_out.at[pl.ds(e0, chunk)])
            pltpu.sync_copy(bias_v, bias_out.at[pl.ds(e0, chunk)])
            return carry

        lax.fori_loop(0, n_chunks, chunk_body, 0)

    return sc_kernel(et_flat, mul_tbl, bias_tbl)


_SIN_C = (6.283182793407033, -41.34141938561704, 81.59613875538135,
          -76.5796878510129, 41.203743633642276, -12.268859940984608)
_INV_2PI = 0.15915494309189535


def _sin_bounded(a):
    r = a * _INV_2PI
    t = r - jnp.floor(r + 0.5)
    u = t * t
    p = jnp.float32(_SIN_C[5])
    for i in (4, 3, 2, 1, 0):
        p = p * u + jnp.float32(_SIN_C[i])
    return p * t


def _bf16_parts(v, n):
    parts = []
    for _ in range(n - 1):
        h = v.astype(jnp.bfloat16)
        parts.append(h)
        v = v - h.astype(jnp.float32)
    parts.append(v.astype(jnp.bfloat16))
    return parts


def _expand(v, sel_bf, n):
    acc = None
    for p in _bf16_parts(v, n):
        d = lax.dot(p, sel_bf, preferred_element_type=jnp.float32)
        acc = d if acc is None else acc + d
    return acc


def _tc_body(x_ref, m_ref, b_ref, s_ref, w_ref, o_ref):
    xb = x_ref[...]
    coef = m_ref[...] / xb
    sel = s_ref[...]
    xa = _expand(xb, sel, 3)
    ca = _expand(coef, sel, 2)
    ba = _expand(b_ref[...], sel, 2)
    o_ref[...] = ca * _sin_bounded(xa * w_ref[...]) + ba


def _tc_basis_call(x2, mul2, bias2, sel, wt, block_rows):
    rows = x2.shape[0]
    grid = (pl.cdiv(rows, block_rows),)
    return pl.pallas_call(
        _tc_body,
        grid=grid,
        in_specs=[
            pl.BlockSpec((block_rows, LANES_PER_ROW), lambda i: (i, 0)),
            pl.BlockSpec((block_rows, LANES_PER_ROW), lambda i: (i, 0)),
            pl.BlockSpec((block_rows, LANES_PER_ROW), lambda i: (i, 0)),
            pl.BlockSpec((LANES_PER_ROW, EXP_COLS), lambda i: (0, 0)),
            pl.BlockSpec((1, EXP_COLS), lambda i: (0, 0)),
        ],
        out_specs=pl.BlockSpec((block_rows, EXP_COLS), lambda i: (i, 0)),
        out_shape=jax.ShapeDtypeStruct((rows, EXP_COLS), jnp.float32),
        compiler_params=pltpu.CompilerParams(
            dimension_semantics=("arbitrary",),
        ),
    )(x2, mul2, bias2, sel, wt)


def kernel(x, edge_types, mul_weight, bias_weight, bessel_weights, prefactor):
    E = x.shape[0]
    nb = bessel_weights.shape[0]

    mul_tbl = mul_weight[:, 0] * prefactor
    bias_tbl = bias_weight[:, 0]
    et_flat = edge_types.reshape(2 * E)

    n_workers = 32
    e_per_worker = E // n_workers
    chunk = 2000
    n_chunks = e_per_worker // chunk
    mul_e, bias_e = _sc_gather_call(
        et_flat, mul_tbl, bias_tbl, chunk, n_chunks, e_per_worker)

    rows = E // LANES_PER_ROW
    x2 = x.reshape(rows, LANES_PER_ROW)
    m2 = mul_e.reshape(rows, LANES_PER_ROW)
    b2 = bias_e.reshape(rows, LANES_PER_ROW)

    col = jnp.arange(EXP_COLS, dtype=jnp.int32) // nb
    sel = (col[None, :] == jnp.arange(LANES_PER_ROW, dtype=jnp.int32)[:, None])
    sel = sel.astype(jnp.bfloat16)
    wt = jnp.tile(bessel_weights, LANES_PER_ROW)[None, :]

    out2 = _tc_basis_call(x2, m2, b2, sel, wt, block_rows=128)
    return out2.reshape(E, nb)


# speedup vs baseline: 169.8114x; 7.6504x over previous
import functools

import jax
import jax.numpy as jnp
from jax import lax
from jax.experimental import pallas as pl
from jax.experimental.pallas import tpu as pltpu
from jax.experimental.pallas import tpu_sc as plsc


def _sc_gather_call(et0, et1, mul_tbl, bias_tbl, chunk, n_chunks, e_per_worker):
    E = et0.shape[0]
    T = mul_tbl.shape[0]
    mesh = plsc.VectorSubcoreMesh(core_axis_name="c", subcore_axis_name="s")

    @functools.partial(
        pl.kernel,
        mesh=mesh,
        compiler_params=pltpu.CompilerParams(needs_layout_passes=False),
        out_type=[
            jax.ShapeDtypeStruct((E,), jnp.float32),
            jax.ShapeDtypeStruct((E,), jnp.float32),
        ],
        scratch_types=[
            pltpu.VMEM((chunk,), jnp.int32),
            pltpu.VMEM((chunk,), jnp.int32),
            pltpu.VMEM((chunk,), jnp.float32),
            pltpu.VMEM((chunk,), jnp.float32),
            pltpu.VMEM((T,), jnp.float32),
            pltpu.VMEM((T,), jnp.float32),
        ],
    )
    def sc_kernel(et0_hbm, et1_hbm, mt_hbm, bt_hbm, mul_out, bias_out,
                  et0_v, et1_v, mul_v, bias_v, mt_v, bt_v):
        nc = 2
        wid = lax.axis_index("s") * nc + lax.axis_index("c")
        pltpu.sync_copy(mt_hbm, mt_v)
        pltpu.sync_copy(bt_hbm, bt_v)
        base_e = wid * e_per_worker

        def chunk_body(ci, carry):
            e0 = base_e + ci * chunk
            pltpu.sync_copy(et0_hbm.at[pl.ds(e0, chunk)], et0_v)
            pltpu.sync_copy(et1_hbm.at[pl.ds(e0, chunk)], et1_v)

            def grp(j, c2):
                o = j * 16
                i0 = et0_v[pl.ds(o, 16)]
                i1 = et1_v[pl.ds(o, 16)]
                mul_v[pl.ds(o, 16)] = (
                    plsc.load_gather(mt_v, [i0]) + plsc.load_gather(mt_v, [i1]))
                bias_v[pl.ds(o, 16)] = (
                    plsc.load_gather(bt_v, [i0]) + plsc.load_gather(bt_v, [i1]))
                return c2

            lax.fori_loop(0, chunk // 16, grp, 0, unroll=4)
            pltpu.sync_copy(mul_v, mul_out.at[pl.ds(e0, chunk)])
            pltpu.sync_copy(bias_v, bias_out.at[pl.ds(e0, chunk)])
            return carry

        lax.fori_loop(0, n_chunks, chunk_body, 0)

    return sc_kernel(et0, et1, mul_tbl, bias_tbl)


_SIN_C = (6.283182793407033, -41.34141938561704, 81.59613875538135,
          -76.5796878510129, 41.203743633642276, -12.268859940984608)


def _tc_body(x_ref, m_ref, b_ref, w_ref, o_ref):
    xb = x_ref[...]
    wr = w_ref[...]
    coef = m_ref[...] / xb
    r = wr * xb
    t = r - jnp.floor(r + 0.5)
    u = t * t
    p = jnp.float32(_SIN_C[5])
    for i in (4, 3, 2, 1, 0):
        p = p * u + jnp.float32(_SIN_C[i])
    o_ref[...] = coef * (p * t) + b_ref[...]


def _tc_basis_call(x2, mul2, bias2, w2, block_cols):
    nb = w2.shape[0]
    E = x2.shape[1]
    grid = (E // block_cols,)
    return pl.pallas_call(
        _tc_body,
        grid=grid,
        in_specs=[
            pl.BlockSpec((1, block_cols), lambda i: (0, i)),
            pl.BlockSpec((1, block_cols), lambda i: (0, i)),
            pl.BlockSpec((1, block_cols), lambda i: (0, i)),
            pl.BlockSpec((nb, 1), lambda i: (0, 0)),
        ],
        out_specs=pl.BlockSpec((nb, block_cols), lambda i: (0, i)),
        out_shape=jax.ShapeDtypeStruct((nb, E), jnp.float32),
        compiler_params=pltpu.CompilerParams(
            dimension_semantics=("arbitrary",),
        ),
    )(x2, mul2, bias2, w2)


def kernel(x, edge_types, mul_weight, bias_weight, bessel_weights, prefactor):
    E = x.shape[0]
    nb = bessel_weights.shape[0]

    mul_tbl = mul_weight[:, 0] * prefactor
    bias_tbl = bias_weight[:, 0]
    et0 = edge_types[:, 0]
    et1 = edge_types[:, 1]

    n_workers = 32
    e_per_worker = E // n_workers
    chunk = 2000
    n_chunks = e_per_worker // chunk
    mul_e, bias_e = _sc_gather_call(
        et0, et1, mul_tbl, bias_tbl, chunk, n_chunks, e_per_worker)

    x2 = x.reshape(1, E)
    m2 = mul_e.reshape(1, E)
    b2 = bias_e.reshape(1, E)
    w2 = (bessel_weights * jnp.float32(1.0 / (2.0 * jnp.pi))).reshape(nb, 1)

    out_t = _tc_basis_call(x2, m2, b2, w2, block_cols=12800)
    return out_t.T
